# Initial kernel scaffold; baseline (speedup 1.0000x reference)
import jax
import jax.numpy as jnp
from jax.experimental import pallas as pl


def kernel(sites, cand0, cand1, width, height, inv_scale_sq):
    raise NotImplementedError("write your pallas kernel here")


# speedup vs baseline: 8.6868x; 8.6868x over previous
import functools

import jax
import jax.numpy as jnp
from jax import lax
from jax.experimental import pallas as pl
from jax.experimental.pallas import tpu as pltpu
from jax.experimental.pallas import tpu_sc as plsc

N_CORES = 2
N_SUBCORES = 16
NW = N_CORES * N_SUBCORES
L = 16


def _build_sc_kernel(n_sites, npix, chunk, groups, n_chunks, shift_w, mask_w):
    mesh = plsc.VectorSubcoreMesh(
        core_axis_name="c", subcore_axis_name="s",
        num_cores=N_CORES, num_subcores=N_SUBCORES)
    per_w = npix // NW

    @functools.partial(
        pl.kernel,
        out_type=jax.ShapeDtypeStruct((npix * 3,), jnp.float32),
        mesh=mesh,
        scratch_types=[
            pltpu.VMEM((n_sites * 5,), jnp.float32),
            pltpu.VMEM((chunk,), jnp.int32),
            pltpu.VMEM((chunk,), jnp.int32),
            pltpu.VMEM((3 * chunk,), jnp.float32),
            pltpu.VMEM((3 * L,), jnp.float32),
        ],
        compiler_params=pltpu.CompilerParams(use_tc_tiling_on_sc=False,
                                             needs_layout_passes=False),
    )
    def sad_sc(sites_hbm, c0_hbm, c1_hbm, par_hbm, out_hbm,
               table_v, c0_v, c1_v, out_v, par_v):
        wid = lax.axis_index("s") * N_CORES + lax.axis_index("c")
        pltpu.sync_copy(sites_hbm, table_v)
        pltpu.sync_copy(par_hbm, par_v)
        inv_w = par_v[pl.ds(0, L)]
        inv_h = par_v[pl.ds(L, L)]
        scale = par_v[pl.ds(2 * L, L)]
        iota = lax.iota(jnp.int32, L)
        iota3 = iota * 3
        base_w = wid * per_w

        def chunk_body(ci, carry):
            base = base_w + ci * chunk
            pltpu.sync_copy(c0_hbm.at[pl.ds(base, chunk)], c0_v)
            pltpu.sync_copy(c1_hbm.at[pl.ds(base, chunk)], c1_v)

            def grp(g, carry2):
                off = g * L
                idx0 = c0_v[pl.ds(off, L)] * 5
                idx1 = c1_v[pl.ds(off, L)] * 5
                x0 = plsc.load_gather(table_v, [idx0])
                y0 = plsc.load_gather(table_v, [idx0 + 1])
                x1 = plsc.load_gather(table_v, [idx1])
                y1 = plsc.load_gather(table_v, [idx1 + 1])
                p = base + off + iota
                px = ((p & mask_w).astype(jnp.float32) + 0.5) * inv_w
                py = (lax.shift_right_logical(p, shift_w).astype(jnp.float32)
                      + 0.5) * inv_h
                dx0 = px - x0
                dy0 = py - y0
                dx1 = px - x1
                dy1 = py - y1
                d0 = dx0 * dx0 + dy0 * dy0
                d1 = dx1 * dx1 + dy1 * dy1
                t = (d1 - d0) * scale
                w = 1.0 / (1.0 + jnp.exp(-t))
                obase = off * 3
                for c in range(3):
                    a = plsc.load_gather(table_v, [idx0 + (2 + c)])
                    b = plsc.load_gather(table_v, [idx1 + (2 + c)])
                    v = b + w * (a - b)
                    plsc.store_scatter(out_v, [obase + iota3 + c], v)
                return carry2

            lax.fori_loop(0, groups, grp, 0)
            pltpu.sync_copy(out_v, out_hbm.at[pl.ds(base * 3, 3 * chunk)])
            return carry

        lax.fori_loop(0, n_chunks, chunk_body, 0)

    return sad_sc


def kernel(sites, cand0, cand1, width, height, inv_scale_sq):
    height_s, width_s = cand0.shape
    n_sites = sites.shape[0]
    npix = height_s * width_s
    assert width_s & (width_s - 1) == 0, "width must be a power of two"
    shift_w = width_s.bit_length() - 1
    mask_w = width_s - 1
    per_w = npix // NW
    chunk = min(8192, per_w)
    groups = chunk // L
    n_chunks = per_w // chunk

    width_f = jnp.asarray(width, dtype=jnp.float32)
    height_f = jnp.asarray(height, dtype=jnp.float32)
    scale_f = jnp.asarray(inv_scale_sq, dtype=jnp.float32)
    params = jnp.concatenate([
        jnp.broadcast_to(1.0 / width_f, (L,)),
        jnp.broadcast_to(1.0 / height_f, (L,)),
        jnp.broadcast_to(scale_f, (L,)),
    ]).astype(jnp.float32)

    sad_sc = _build_sc_kernel(n_sites, npix, chunk, groups, n_chunks,
                              shift_w, mask_w)
    out_flat = sad_sc(sites.reshape(n_sites * 5), cand0.reshape(npix),
                      cand1.reshape(npix), params)
    return out_flat.reshape(height_s, width_s, 3)


# speedup vs baseline: 9.6492x; 1.1108x over previous
import functools

import jax
import jax.numpy as jnp
from jax import lax
from jax.experimental import pallas as pl
from jax.experimental.pallas import tpu as pltpu
from jax.experimental.pallas import tpu_sc as plsc

N_CORES = 2
N_SUBCORES = 16
NW = N_CORES * N_SUBCORES
L = 16


def _build_sc_kernel(n_sites, npix, chunk, groups, n_chunks, shift_w, mask_w):
    mesh = plsc.VectorSubcoreMesh(
        core_axis_name="c", subcore_axis_name="s",
        num_cores=N_CORES, num_subcores=N_SUBCORES)
    per_w = npix // NW

    @functools.partial(
        pl.kernel,
        out_type=jax.ShapeDtypeStruct((npix * 3,), jnp.float32),
        mesh=mesh,
        scratch_types=[
            pltpu.VMEM((n_sites * 5,), jnp.float32),
            pltpu.VMEM((chunk,), jnp.int32),
            pltpu.VMEM((chunk,), jnp.int32),
            pltpu.VMEM((3 * chunk,), jnp.float32),
            pltpu.VMEM((3 * L,), jnp.float32),
        ],
        compiler_params=pltpu.CompilerParams(use_tc_tiling_on_sc=False,
                                             needs_layout_passes=False),
    )
    def sad_sc(sites_hbm, c0_hbm, c1_hbm, par_hbm, out_hbm,
               table_v, c0_v, c1_v, out_v, par_v):
        wid = lax.axis_index("s") * N_CORES + lax.axis_index("c")
        pltpu.sync_copy(sites_hbm, table_v)
        pltpu.sync_copy(par_hbm, par_v)
        inv_w = par_v[pl.ds(0, L)]
        inv_h = par_v[pl.ds(L, L)]
        scale = par_v[pl.ds(2 * L, L)]
        iota = lax.iota(jnp.int32, L)
        iota3 = iota * 3
        base_w = wid * per_w

        def chunk_body(ci, carry):
            base = base_w + ci * chunk
            pltpu.sync_copy(c0_hbm.at[pl.ds(base, chunk)], c0_v)
            pltpu.sync_copy(c1_hbm.at[pl.ds(base, chunk)], c1_v)

            @plsc.parallel_loop(0, groups, 1, unroll=4)
            def grp(g):
                off = g * L
                idx0 = c0_v[pl.ds(off, L)] * 5
                idx1 = c1_v[pl.ds(off, L)] * 5
                x0 = plsc.load_gather(table_v, [idx0])
                y0 = plsc.load_gather(table_v, [idx0 + 1])
                x1 = plsc.load_gather(table_v, [idx1])
                y1 = plsc.load_gather(table_v, [idx1 + 1])
                p = base + off + iota
                px = ((p & mask_w).astype(jnp.float32) + 0.5) * inv_w
                py = (lax.shift_right_logical(p, shift_w).astype(jnp.float32)
                      + 0.5) * inv_h
                dx0 = px - x0
                dy0 = py - y0
                dx1 = px - x1
                dy1 = py - y1
                d0 = dx0 * dx0 + dy0 * dy0
                d1 = dx1 * dx1 + dy1 * dy1
                t = (d1 - d0) * scale
                w = 1.0 / (1.0 + jnp.exp(-t))
                obase = off * 3
                for c in range(3):
                    a = plsc.load_gather(table_v, [idx0 + (2 + c)])
                    b = plsc.load_gather(table_v, [idx1 + (2 + c)])
                    v = b + w * (a - b)
                    plsc.store_scatter(out_v, [obase + iota3 + c], v)

            pltpu.sync_copy(out_v, out_hbm.at[pl.ds(base * 3, 3 * chunk)])
            return carry

        lax.fori_loop(0, n_chunks, chunk_body, 0)

    return sad_sc


def kernel(sites, cand0, cand1, width, height, inv_scale_sq):
    height_s, width_s = cand0.shape
    n_sites = sites.shape[0]
    npix = height_s * width_s
    assert width_s & (width_s - 1) == 0, "width must be a power of two"
    shift_w = width_s.bit_length() - 1
    mask_w = width_s - 1
    per_w = npix // NW
    chunk = min(8192, per_w)
    groups = chunk // L
    n_chunks = per_w // chunk

    width_f = jnp.asarray(width, dtype=jnp.float32)
    height_f = jnp.asarray(height, dtype=jnp.float32)
    scale_f = jnp.asarray(inv_scale_sq, dtype=jnp.float32)
    params = jnp.concatenate([
        jnp.broadcast_to(1.0 / width_f, (L,)),
        jnp.broadcast_to(1.0 / height_f, (L,)),
        jnp.broadcast_to(scale_f, (L,)),
    ]).astype(jnp.float32)

    sad_sc = _build_sc_kernel(n_sites, npix, chunk, groups, n_chunks,
                              shift_w, mask_w)
    out_flat = sad_sc(sites.reshape(n_sites * 5), cand0.reshape(npix),
                      cand1.reshape(npix), params)
    return out_flat.reshape(height_s, width_s, 3)


# speedup vs baseline: 67.2369x; 6.9681x over previous
import functools

import jax
import jax.numpy as jnp
from jax import lax
from jax.experimental import pallas as pl
from jax.experimental.pallas import tpu as pltpu
from jax.experimental.pallas import tpu_sc as plsc

N_CORES = 2
N_SUBCORES = 16
NW = N_CORES * N_SUBCORES
L = 16


def _build_sc_kernel(n_sites, npix, chunk, groups, n_chunks, shift_w, mask_w):
    mesh = plsc.VectorSubcoreMesh(
        core_axis_name="c", subcore_axis_name="s",
        num_cores=N_CORES, num_subcores=N_SUBCORES)
    per_w = npix // NW

    @functools.partial(
        pl.kernel,
        out_type=jax.ShapeDtypeStruct((npix * 3,), jnp.float32),
        mesh=mesh,
        scratch_types=[
            pltpu.VMEM((n_sites * 5,), jnp.float32),
            pltpu.VMEM((chunk,), jnp.int32),
            pltpu.VMEM((chunk,), jnp.int32),
            [pltpu.VMEM((chunk,), jnp.float32) for _ in range(3)],
            pltpu.VMEM((3 * L,), jnp.float32),
        ],
        compiler_params=pltpu.CompilerParams(use_tc_tiling_on_sc=False,
                                             needs_layout_passes=False),
    )
    def sad_sc(sites_hbm, c0_hbm, c1_hbm, par_hbm, out_hbm,
               table_v, c0_v, c1_v, out_v, par_v):
        wid = lax.axis_index("s") * N_CORES + lax.axis_index("c")
        pltpu.sync_copy(sites_hbm, table_v)
        pltpu.sync_copy(par_hbm, par_v)
        inv_w = par_v[pl.ds(0, L)]
        inv_h = par_v[pl.ds(L, L)]
        scale = par_v[pl.ds(2 * L, L)]
        iota = lax.iota(jnp.int32, L)
        base_w = wid * per_w

        def chunk_body(ci, carry):
            base = base_w + ci * chunk
            pltpu.sync_copy(c0_hbm.at[pl.ds(base, chunk)], c0_v)
            pltpu.sync_copy(c1_hbm.at[pl.ds(base, chunk)], c1_v)

            @plsc.parallel_loop(0, groups, 1, unroll=4)
            def grp(g):
                off = g * L
                idx0 = c0_v[pl.ds(off, L)] * 5
                idx1 = c1_v[pl.ds(off, L)] * 5
                x0 = plsc.load_gather(table_v, [idx0])
                y0 = plsc.load_gather(table_v, [idx0 + 1])
                x1 = plsc.load_gather(table_v, [idx1])
                y1 = plsc.load_gather(table_v, [idx1 + 1])
                p = base + off + iota
                px = ((p & mask_w).astype(jnp.float32) + 0.5) * inv_w
                py = (lax.shift_right_logical(p, shift_w).astype(jnp.float32)
                      + 0.5) * inv_h
                dx0 = px - x0
                dy0 = py - y0
                dx1 = px - x1
                dy1 = py - y1
                d0 = dx0 * dx0 + dy0 * dy0
                d1 = dx1 * dx1 + dy1 * dy1
                t = (d1 - d0) * scale
                w = 1.0 / (1.0 + jnp.exp(-t))
                for c in range(3):
                    a = plsc.load_gather(table_v, [idx0 + (2 + c)])
                    b = plsc.load_gather(table_v, [idx1 + (2 + c)])
                    out_v[c][pl.ds(off, L)] = b + w * (a - b)

            for c in range(3):
                pltpu.sync_copy(out_v[c], out_hbm.at[pl.ds(c * npix + base, chunk)])
            return carry

        lax.fori_loop(0, n_chunks, chunk_body, 0)

    return sad_sc


def kernel(sites, cand0, cand1, width, height, inv_scale_sq):
    height_s, width_s = cand0.shape
    n_sites = sites.shape[0]
    npix = height_s * width_s
    assert width_s & (width_s - 1) == 0, "width must be a power of two"
    shift_w = width_s.bit_length() - 1
    mask_w = width_s - 1
    per_w = npix // NW
    chunk = min(8192, per_w)
    groups = chunk // L
    n_chunks = per_w // chunk

    width_f = jnp.asarray(width, dtype=jnp.float32)
    height_f = jnp.asarray(height, dtype=jnp.float32)
    scale_f = jnp.asarray(inv_scale_sq, dtype=jnp.float32)
    params = jnp.concatenate([
        jnp.broadcast_to(1.0 / width_f, (L,)),
        jnp.broadcast_to(1.0 / height_f, (L,)),
        jnp.broadcast_to(scale_f, (L,)),
    ]).astype(jnp.float32)

    sad_sc = _build_sc_kernel(n_sites, npix, chunk, groups, n_chunks,
                              shift_w, mask_w)
    out_flat = sad_sc(sites.reshape(n_sites * 5), cand0.reshape(npix),
                      cand1.reshape(npix), params)
    return out_flat.reshape(3, height_s, width_s).transpose(1, 2, 0)


# speedup vs baseline: 80.4355x; 1.1963x over previous
import functools

import jax
import jax.numpy as jnp
from jax import lax
from jax.experimental import pallas as pl
from jax.experimental.pallas import tpu as pltpu
from jax.experimental.pallas import tpu_sc as plsc

N_CORES = 2
N_SUBCORES = 16
NW = N_CORES * N_SUBCORES
L = 16


def _build_sc_kernel(n_sites, npix, chunk, groups, n_chunks, shift_w, mask_w):
    mesh = plsc.VectorSubcoreMesh(
        core_axis_name="c", subcore_axis_name="s",
        num_cores=N_CORES, num_subcores=N_SUBCORES)
    per_w = npix // NW

    @functools.partial(
        pl.kernel,
        out_type=jax.ShapeDtypeStruct((npix * 3,), jnp.float32),
        mesh=mesh,
        scratch_types=[
            pltpu.VMEM((n_sites * 5,), jnp.float32),
            pltpu.VMEM((chunk,), jnp.int32),
            pltpu.VMEM((chunk,), jnp.int32),
            [pltpu.VMEM((chunk,), jnp.float32) for _ in range(3)],
            pltpu.VMEM((3 * L,), jnp.float32),
        ],
        compiler_params=pltpu.CompilerParams(use_tc_tiling_on_sc=False,
                                             needs_layout_passes=False),
    )
    def sad_sc(sites_hbm, c0_hbm, c1_hbm, par_hbm, out_hbm,
               table_v, c0_v, c1_v, out_v, par_v):
        wid = lax.axis_index("s") * N_CORES + lax.axis_index("c")
        pltpu.sync_copy(sites_hbm, table_v)
        pltpu.sync_copy(par_hbm, par_v)
        inv_w = par_v[pl.ds(0, L)]
        inv_h = par_v[pl.ds(L, L)]
        scale = par_v[pl.ds(2 * L, L)]
        iota = lax.iota(jnp.int32, L)
        base_w = wid * per_w

        def chunk_body(ci, carry):
            base = base_w + ci * chunk
            pltpu.sync_copy(c0_hbm.at[pl.ds(base, chunk)], c0_v)
            pltpu.sync_copy(c1_hbm.at[pl.ds(base, chunk)], c1_v)

            row0 = lax.shift_right_logical(base, shift_w)

            @plsc.parallel_loop(0, groups, 1, unroll=4)
            def grp(g):
                off = g * L
                cc = lax.shift_right_logical(g, 6)
                rem = g & 63
                r = lax.shift_right_logical(rem, 3)
                l0 = (rem & 7) * L
                qoff = r * (mask_w + 1) + cc * 128 + l0
                idx0 = c0_v[pl.ds(qoff, L)] * 5
                idx1 = c1_v[pl.ds(qoff, L)] * 5
                x0 = plsc.load_gather(table_v, [idx0])
                y0 = plsc.load_gather(table_v, [idx0 + 1])
                x1 = plsc.load_gather(table_v, [idx1])
                y1 = plsc.load_gather(table_v, [idx1 + 1])
                xi = cc * 128 + l0 + iota
                px = (xi.astype(jnp.float32) + 0.5) * inv_w
                yi = jnp.full((L,), row0 + r, jnp.int32)
                py = (yi.astype(jnp.float32) + 0.5) * inv_h
                dx0 = px - x0
                dy0 = py - y0
                dx1 = px - x1
                dy1 = py - y1
                d0 = dx0 * dx0 + dy0 * dy0
                d1 = dx1 * dx1 + dy1 * dy1
                t = (d1 - d0) * scale
                w = 1.0 / (1.0 + jnp.exp(-t))
                for c in range(3):
                    a = plsc.load_gather(table_v, [idx0 + (2 + c)])
                    b = plsc.load_gather(table_v, [idx1 + (2 + c)])
                    out_v[c][pl.ds(off, L)] = b + w * (a - b)

            for c in range(3):
                pltpu.sync_copy(out_v[c], out_hbm.at[pl.ds(c * npix + base, chunk)])
            return carry

        lax.fori_loop(0, n_chunks, chunk_body, 0)

    return sad_sc


def kernel(sites, cand0, cand1, width, height, inv_scale_sq):
    height_s, width_s = cand0.shape
    n_sites = sites.shape[0]
    npix = height_s * width_s
    assert width_s & (width_s - 1) == 0, "width must be a power of two"
    shift_w = width_s.bit_length() - 1
    mask_w = width_s - 1
    per_w = npix // NW
    chunk = min(8192, per_w)
    groups = chunk // L
    n_chunks = per_w // chunk

    width_f = jnp.asarray(width, dtype=jnp.float32)
    height_f = jnp.asarray(height, dtype=jnp.float32)
    scale_f = jnp.asarray(inv_scale_sq, dtype=jnp.float32)
    params = jnp.concatenate([
        jnp.broadcast_to(1.0 / width_f, (L,)),
        jnp.broadcast_to(1.0 / height_f, (L,)),
        jnp.broadcast_to(scale_f, (L,)),
    ]).astype(jnp.float32)

    sad_sc = _build_sc_kernel(n_sites, npix, chunk, groups, n_chunks,
                              shift_w, mask_w)
    out_flat = sad_sc(sites.reshape(n_sites * 5), cand0.reshape(npix),
                      cand1.reshape(npix), params)
    out5 = out_flat.reshape(3, height_s // 8, width_s // 128, 8, 128)
    return out5.transpose(1, 3, 2, 4, 0).reshape(height_s, width_s, 3)
